# Initial kernel scaffold; baseline (speedup 1.0000x reference)
import jax
import jax.numpy as jnp
from jax.experimental import pallas as pl


def kernel(src0, src1, src2, src3, pos0, pos1, pos2, pos3, level_embed, Wv, bv, Woff, boff, Wattn, battn, Wout, bout, g1, be1, W1, b1, W2, b2, g2, be2):
    raise NotImplementedError("write your pallas kernel here")


# speedup vs baseline: 2337.1291x; 2337.1291x over previous
import dataclasses
import functools

import jax
import jax.numpy as jnp
import numpy as np
from jax import lax
from jax.experimental import pallas as pl
from jax.experimental.pallas import tpu as pltpu
from jax.experimental.pallas import tpu_sc as plsc

BB = 2
DM = 256
NH = 8
DH = 32
NLAY = 6
DF = 1024
NL = 4
NP = 4
SH = [(64, 64), (32, 32), (16, 16), (8, 8)]
LQ = sum(h * w for h, w in SH)
NR = BB * LQ * NH
BLK = 680
NBLK = LQ // BLK
NW = 32
RCH = 16
ROWS_W = NR // NW
NIT = ROWS_W // RCH


def _build_static():
    lane = np.arange(128)
    lvl = (lane % 16) // 4
    wl = np.array([w for (h, w) in SH], np.int32)[lvl]
    hl = np.array([h for (h, w) in SH], np.int32)[lvl]
    base = np.array([0, 4096, 5120, 5376], np.int32)[lvl]
    hlane = (lane // 16).astype(np.int32)
    rx, ry = [], []
    for (h, w) in SH:
        yy, xx = np.meshgrid(np.linspace(0.5, h - 0.5, h), np.linspace(0.5, w - 0.5, w), indexing='ij')
        rx.append((xx / w).reshape(-1))
        ry.append((yy / h).reshape(-1))
    rx = np.concatenate(rx)
    ry = np.concatenate(ry)
    refx = (rx[:, None] * wl[None, :].astype(np.float64) - 0.5).astype(np.float32)
    refy = (ry[:, None] * hl[None, :].astype(np.float64) - 0.5).astype(np.float32)
    msum = (lane[:, None] // 16 == lane[None, :] // 16).astype(np.float32)
    return (wl[None, :], hl[None, :], base[None, :], hlane[None, :], refx, refy, msum)


_WL, _HL, _BASE, _HLANE, _REFX, _REFY, _MSUM = _build_static()


def _pre_body(out_ref, pos_ref, Wv_ref, bv_ref, Woff_ref, boff_ref, Wattn_ref,
              battn_ref, refx_ref, refy_ref, wl_ref, hl_ref, base_ref, hlane_ref,
              msum_ref, val_ref, i0_ref, i1_ref, i2_ref, i3_ref, w0_ref, w1_ref,
              w2_ref, w3_ref):
    b = pl.program_id(0)
    x_in = out_ref[0]
    q = x_in + pos_ref[0]
    val_ref[0] = jnp.dot(x_in, Wv_ref[...], preferred_element_type=jnp.float32) + bv_ref[...]
    off = jnp.dot(q, Woff_ref[...], preferred_element_type=jnp.float32) + boff_ref[...]
    e = jnp.exp(jnp.dot(q, Wattn_ref[...], preferred_element_type=jnp.float32) + battn_ref[...])
    p = e / jnp.dot(e, msum_ref[...], preferred_element_type=jnp.float32)
    x = refx_ref[...] + off[:, :128]
    y = refy_ref[...] + off[:, 128:]
    x0f = jnp.floor(x)
    y0f = jnp.floor(y)
    fx = x - x0f
    fy = y - y0f
    x0 = x0f.astype(jnp.int32)
    y0 = y0f.astype(jnp.int32)
    wl = wl_ref[...]
    hl = hl_ref[...]
    basel = base_ref[...]
    hlane = hlane_ref[...]
    iouts = (i0_ref, i1_ref, i2_ref, i3_ref)
    wouts = (w0_ref, w1_ref, w2_ref, w3_ref)
    k = 0
    for dy in (0, 1):
        for dx in (0, 1):
            xc = x0 + dx
            yc = y0 + dy
            valid = (xc >= 0) & (xc < wl) & (yc >= 0) & (yc < hl)
            wx = fx if dx else 1.0 - fx
            wy = fy if dy else 1.0 - fy
            wt = jnp.where(valid, p * wx * wy, 0.0)
            xcc = jnp.clip(xc, 0, wl - 1)
            ycc = jnp.clip(yc, 0, hl - 1)
            loc = basel + ycc * wl + xcc
            row = ((b * LQ + loc) << 3) + hlane
            iouts[k][0] = row
            wouts[k][0] = wt
            k += 1


def _pre_layer(out, pos, Wv, bv, Woff, boff, Wattn, battn):
    f32 = jnp.float32
    i32 = jnp.int32
    outs = pl.pallas_call(
        _pre_body,
        grid=(BB, NBLK),
        in_specs=[
            pl.BlockSpec((1, BLK, DM), lambda b, j: (b, j, 0)),
            pl.BlockSpec((1, BLK, DM), lambda b, j: (b, j, 0)),
            pl.BlockSpec((DM, DM), lambda b, j: (0, 0)),
            pl.BlockSpec((1, DM), lambda b, j: (0, 0)),
            pl.BlockSpec((DM, DM), lambda b, j: (0, 0)),
            pl.BlockSpec((1, DM), lambda b, j: (0, 0)),
            pl.BlockSpec((DM, 128), lambda b, j: (0, 0)),
            pl.BlockSpec((1, 128), lambda b, j: (0, 0)),
            pl.BlockSpec((BLK, 128), lambda b, j: (j, 0)),
            pl.BlockSpec((BLK, 128), lambda b, j: (j, 0)),
            pl.BlockSpec((1, 128), lambda b, j: (0, 0)),
            pl.BlockSpec((1, 128), lambda b, j: (0, 0)),
            pl.BlockSpec((1, 128), lambda b, j: (0, 0)),
            pl.BlockSpec((1, 128), lambda b, j: (0, 0)),
            pl.BlockSpec((128, 128), lambda b, j: (0, 0)),
        ],
        out_specs=[
            pl.BlockSpec((1, BLK, DM), lambda b, j: (b, j, 0)),
        ] + [pl.BlockSpec((1, BLK, 128), lambda b, j: (b, j, 0))] * 8,
        out_shape=[jax.ShapeDtypeStruct((BB, LQ, DM), f32)]
        + [jax.ShapeDtypeStruct((BB, LQ, 128), i32)] * 4
        + [jax.ShapeDtypeStruct((BB, LQ, 128), f32)] * 4,
    )(out, pos, Wv, bv[None, :], Woff, boff[None, :], Wattn, battn[None, :],
      jnp.asarray(_REFX), jnp.asarray(_REFY), jnp.asarray(_WL), jnp.asarray(_HL),
      jnp.asarray(_BASE), jnp.asarray(_HLANE), jnp.asarray(_MSUM))
    return outs[0], outs[1:5], outs[5:9]


def _sc_body(table, i0, i1, i2, i3, w0, w1, w2, w3, out,
                       ib0, ib1, ib2, ib3, wb0, wb1, wb2, wb3,
                       gb0, gb1, gb2, gb3, ob, sem):
    ihbm = (i0, i1, i2, i3)
    whbm = (w0, w1, w2, w3)
    ib = (ib0, ib1, ib2, ib3)
    wb = (wb0, wb1, wb2, wb3)
    gb = (gb0, gb1, gb2, gb3)
    wid = lax.axis_index("s") * 2 + lax.axis_index("c")

    @pl.loop(0, NIT)
    def _(it):
        r0 = wid * ROWS_W + it * RCH
        for t in range(4):
            pltpu.sync_copy(ihbm[t].at[pl.ds(r0 * 16, RCH * 16)], ib[t])
            pltpu.sync_copy(whbm[t].at[pl.ds(r0 * 16, RCH * 16)], wb[t])
        copies = [pltpu.async_copy(table.at[ib[t]], gb[t], sem) for t in range(4)]
        for c in copies:
            c.wait()

        @pl.loop(0, RCH)
        def _(r):
            def tapj(j, accs):
                a0, a1 = accs
                k = r * 16 + j
                for t in range(4):
                    wv = plsc.load_gather(wb[t], [jnp.full((16,), k, jnp.int32)])
                    a0 = a0 + wv * gb[t][k, 0:16]
                    a1 = a1 + wv * gb[t][k, 16:32]
                return (a0, a1)

            z = jnp.zeros((16,), jnp.float32)
            a0, a1 = lax.fori_loop(0, 16, tapj, (z, z))
            ob[r, 0:16] = a0
            ob[r, 16:32] = a1

        pltpu.sync_copy(ob, out.at[pl.ds(r0, RCH)])


@functools.lru_cache(maxsize=1)
def _get_sc_kernel():
    mesh = plsc.VectorSubcoreMesh(core_axis_name="c", subcore_axis_name="s")
    cp = pltpu.CompilerParams()
    fields = pltpu.CompilerParams.__dataclass_fields__
    if "needs_layout_passes" in fields:
        cp = dataclasses.replace(cp, needs_layout_passes=False)
    if "use_tc_tiling_on_sc" in fields:
        cp = dataclasses.replace(cp, use_tc_tiling_on_sc=False)
    return pl.kernel(
        _sc_body,
        mesh=mesh,
        compiler_params=cp,
        out_type=jax.ShapeDtypeStruct((NR, DH), jnp.float32),
        scratch_types=(
            [pltpu.VMEM((RCH * 16,), jnp.int32) for _ in range(4)]
            + [pltpu.VMEM((RCH * 16,), jnp.float32) for _ in range(4)]
            + [pltpu.VMEM((RCH * 16, DH), jnp.float32) for _ in range(4)]
            + [pltpu.VMEM((RCH, DH), jnp.float32), pltpu.SemaphoreType.DMA]
        ),
    )


def _sc_combine(value, idxs, wts):
    table = value.reshape(NR, DH)
    args = [table]
    for t in range(4):
        args.append(idxs[t].reshape(NR * 16))
    for t in range(4):
        args.append(wts[t].reshape(NR * 16))
    o = _get_sc_kernel()(*args)
    return o.reshape(BB, LQ, DM)


def _ln(x, g, b):
    mu = jnp.mean(x, axis=-1, keepdims=True)
    var = jnp.mean((x - mu) ** 2, axis=-1, keepdims=True)
    return (x - mu) * lax.rsqrt(var + 1e-5) * g + b


def _post_body(out_ref, o_ref, Wout_ref, bout_ref, g1_ref, be1_ref, W1_ref,
               b1_ref, W2_ref, b2_ref, g2_ref, be2_ref, new_ref):
    attn = jnp.dot(o_ref[0], Wout_ref[...], preferred_element_type=jnp.float32) + bout_ref[...]
    h1 = _ln(out_ref[0] + attn, g1_ref[...], be1_ref[...])
    a1 = jnp.maximum(jnp.dot(h1, W1_ref[...], preferred_element_type=jnp.float32) + b1_ref[...], 0.0)
    ffn = jnp.dot(a1, W2_ref[...], preferred_element_type=jnp.float32) + b2_ref[...]
    new_ref[0] = _ln(h1 + ffn, g2_ref[...], be2_ref[...])


def _post_layer(out, o, Wout, bout, g1, be1, W1, b1, W2, b2, g2, be2):
    return pl.pallas_call(
        _post_body,
        grid=(BB, NBLK),
        in_specs=[
            pl.BlockSpec((1, BLK, DM), lambda b, j: (b, j, 0)),
            pl.BlockSpec((1, BLK, DM), lambda b, j: (b, j, 0)),
            pl.BlockSpec((DM, DM), lambda b, j: (0, 0)),
            pl.BlockSpec((1, DM), lambda b, j: (0, 0)),
            pl.BlockSpec((1, DM), lambda b, j: (0, 0)),
            pl.BlockSpec((1, DM), lambda b, j: (0, 0)),
            pl.BlockSpec((DM, DF), lambda b, j: (0, 0)),
            pl.BlockSpec((1, DF), lambda b, j: (0, 0)),
            pl.BlockSpec((DF, DM), lambda b, j: (0, 0)),
            pl.BlockSpec((1, DM), lambda b, j: (0, 0)),
            pl.BlockSpec((1, DM), lambda b, j: (0, 0)),
            pl.BlockSpec((1, DM), lambda b, j: (0, 0)),
        ],
        out_specs=pl.BlockSpec((1, BLK, DM), lambda b, j: (b, j, 0)),
        out_shape=jax.ShapeDtypeStruct((BB, LQ, DM), jnp.float32),
    )(out, o, Wout, bout[None, :], g1[None, :], be1[None, :], W1, b1[None, :],
      W2, b2[None, :], g2[None, :], be2[None, :])


def kernel(src0, src1, src2, src3, pos0, pos1, pos2, pos3, level_embed, Wv, bv,
           Woff, boff, Wattn, battn, Wout, bout, g1, be1, W1, b1, W2, b2, g2, be2):
    srcs = [src0, src1, src2, src3]
    poss = [pos0, pos1, pos2, pos3]
    src_f = []
    pos_f = []
    for lvl in range(NL):
        b, c, h, w = srcs[lvl].shape
        src_f.append(srcs[lvl].reshape(b, c, h * w).transpose(0, 2, 1))
        pos_f.append(poss[lvl].reshape(b, c, h * w).transpose(0, 2, 1) + level_embed[lvl][None, None, :])
    src = jnp.concatenate(src_f, 1)
    pos = jnp.concatenate(pos_f, 1)

    Woffp = jnp.moveaxis(Woff.reshape(NLAY, DM, NH, NL, NP, 2), -1, 2).reshape(NLAY, DM, 2 * 128)
    boffp = jnp.moveaxis(boff.reshape(NLAY, NH, NL, NP, 2), -1, 1).reshape(NLAY, 2 * 128)

    out = src
    for i in range(NLAY):
        val, idxs, wts = _pre_layer(out, pos, Wv[i], bv[i], Woffp[i], boffp[i], Wattn[i], battn[i])
        o = _sc_combine(val, idxs, wts)
        out = _post_layer(out, o, Wout[i], bout[i], g1[i], be1[i], W1[i], b1[i], W2[i], b2[i], g2[i], be2[i])
    return out


# speedup vs baseline: 5877.8132x; 2.5150x over previous
import dataclasses
import functools

import jax
import jax.numpy as jnp
import numpy as np
from jax import lax
from jax.experimental import pallas as pl
from jax.experimental.pallas import tpu as pltpu
from jax.experimental.pallas import tpu_sc as plsc

BB = 2
DM = 256
NH = 8
DH = 32
NLAY = 6
DF = 1024
NL = 4
NP = 4
SH = [(64, 64), (32, 32), (16, 16), (8, 8)]
LQ = sum(h * w for h, w in SH)
NR = BB * LQ * NH
BLK = 680
NBLK = LQ // BLK
NW = 32
RCH = 16
ROWS_W = NR // NW
NIT = ROWS_W // RCH


def _build_static():
    lane = np.arange(128)
    lvl = (lane % 16) // 4
    wl = np.array([w for (h, w) in SH], np.int32)[lvl]
    hl = np.array([h for (h, w) in SH], np.int32)[lvl]
    base = np.array([0, 4096, 5120, 5376], np.int32)[lvl]
    hlane = (lane // 16).astype(np.int32)
    rx, ry = [], []
    for (h, w) in SH:
        yy, xx = np.meshgrid(np.linspace(0.5, h - 0.5, h), np.linspace(0.5, w - 0.5, w), indexing='ij')
        rx.append((xx / w).reshape(-1))
        ry.append((yy / h).reshape(-1))
    rx = np.concatenate(rx)
    ry = np.concatenate(ry)
    refx = (rx[:, None] * wl[None, :].astype(np.float64) - 0.5).astype(np.float32)
    refy = (ry[:, None] * hl[None, :].astype(np.float64) - 0.5).astype(np.float32)
    msum = (lane[:, None] // 16 == lane[None, :] // 16).astype(np.float32)
    return (wl[None, :], hl[None, :], base[None, :], hlane[None, :], refx, refy, msum)


_WL, _HL, _BASE, _HLANE, _REFX, _REFY, _MSUM = _build_static()


def _pre_body(out_ref, pos_ref, Wv_ref, bv_ref, Woff_ref, boff_ref, Wattn_ref,
              battn_ref, refx_ref, refy_ref, wl_ref, hl_ref, base_ref, hlane_ref,
              msum_ref, val_ref, i0_ref, i1_ref, i2_ref, i3_ref, w0_ref, w1_ref,
              w2_ref, w3_ref):
    b = pl.program_id(0)
    x_in = out_ref[0]
    q = x_in + pos_ref[0]
    val_ref[0] = jnp.dot(x_in, Wv_ref[...], preferred_element_type=jnp.float32) + bv_ref[...]
    off = jnp.dot(q, Woff_ref[...], preferred_element_type=jnp.float32) + boff_ref[...]
    e = jnp.exp(jnp.dot(q, Wattn_ref[...], preferred_element_type=jnp.float32) + battn_ref[...])
    p = e / jnp.dot(e, msum_ref[...], preferred_element_type=jnp.float32)
    x = refx_ref[...] + off[:, :128]
    y = refy_ref[...] + off[:, 128:]
    x0f = jnp.floor(x)
    y0f = jnp.floor(y)
    fx = x - x0f
    fy = y - y0f
    x0 = x0f.astype(jnp.int32)
    y0 = y0f.astype(jnp.int32)
    wl = wl_ref[...]
    hl = hl_ref[...]
    basel = base_ref[...]
    hlane = hlane_ref[...]
    iouts = (i0_ref, i1_ref, i2_ref, i3_ref)
    wouts = (w0_ref, w1_ref, w2_ref, w3_ref)
    k = 0
    for dy in (0, 1):
        for dx in (0, 1):
            xc = x0 + dx
            yc = y0 + dy
            valid = (xc >= 0) & (xc < wl) & (yc >= 0) & (yc < hl)
            wx = fx if dx else 1.0 - fx
            wy = fy if dy else 1.0 - fy
            wt = jnp.where(valid, p * wx * wy, 0.0)
            xcc = jnp.clip(xc, 0, wl - 1)
            ycc = jnp.clip(yc, 0, hl - 1)
            loc = basel + ycc * wl + xcc
            row = ((b * LQ + loc) << 3) + hlane
            iouts[k][0] = row
            wouts[k][0] = wt
            k += 1


def _pre_layer(out, pos, Wv, bv, Woff, boff, Wattn, battn):
    f32 = jnp.float32
    i32 = jnp.int32
    outs = pl.pallas_call(
        _pre_body,
        grid=(BB, NBLK),
        in_specs=[
            pl.BlockSpec((1, BLK, DM), lambda b, j: (b, j, 0)),
            pl.BlockSpec((1, BLK, DM), lambda b, j: (b, j, 0)),
            pl.BlockSpec((DM, DM), lambda b, j: (0, 0)),
            pl.BlockSpec((1, DM), lambda b, j: (0, 0)),
            pl.BlockSpec((DM, DM), lambda b, j: (0, 0)),
            pl.BlockSpec((1, DM), lambda b, j: (0, 0)),
            pl.BlockSpec((DM, 128), lambda b, j: (0, 0)),
            pl.BlockSpec((1, 128), lambda b, j: (0, 0)),
            pl.BlockSpec((BLK, 128), lambda b, j: (j, 0)),
            pl.BlockSpec((BLK, 128), lambda b, j: (j, 0)),
            pl.BlockSpec((1, 128), lambda b, j: (0, 0)),
            pl.BlockSpec((1, 128), lambda b, j: (0, 0)),
            pl.BlockSpec((1, 128), lambda b, j: (0, 0)),
            pl.BlockSpec((1, 128), lambda b, j: (0, 0)),
            pl.BlockSpec((128, 128), lambda b, j: (0, 0)),
        ],
        out_specs=[
            pl.BlockSpec((1, BLK, DM), lambda b, j: (b, j, 0)),
        ] + [pl.BlockSpec((1, BLK, 128), lambda b, j: (b, j, 0))] * 8,
        out_shape=[jax.ShapeDtypeStruct((BB, LQ, DM), f32)]
        + [jax.ShapeDtypeStruct((BB, LQ, 128), i32)] * 4
        + [jax.ShapeDtypeStruct((BB, LQ, 128), f32)] * 4,
    )(out, pos, Wv, bv[None, :], Woff, boff[None, :], Wattn, battn[None, :],
      jnp.asarray(_REFX), jnp.asarray(_REFY), jnp.asarray(_WL), jnp.asarray(_HL),
      jnp.asarray(_BASE), jnp.asarray(_HLANE), jnp.asarray(_MSUM))
    return outs[0], outs[1:5], outs[5:9]


def _sc_body(table, i0, i1, i2, i3, w0, w1, w2, w3, out, *scr):
    ihbm = (i0, i1, i2, i3)
    whbm = (w0, w1, w2, w3)
    ib = (scr[0:4], scr[4:8])
    wb = (scr[8:12], scr[12:16])
    gb = (scr[16:20], scr[20:24])
    ob = scr[24]
    semi = (scr[25], scr[26])
    semg = (scr[27], scr[28])
    wid = lax.axis_index("s") * 2 + lax.axis_index("c")
    base = wid * ROWS_W

    def fire_loads(it, b):
        r0 = (base + it * RCH) * 16
        for t in range(4):
            pltpu.make_async_copy(ihbm[t].at[pl.ds(r0, RCH * 16)], ib[b][t], semi[b]).start()
            pltpu.make_async_copy(whbm[t].at[pl.ds(r0, RCH * 16)], wb[b][t], semi[b]).start()

    def wait_loads(it, b):
        r0 = (base + it * RCH) * 16
        for t in range(4):
            pltpu.make_async_copy(ihbm[t].at[pl.ds(r0, RCH * 16)], ib[b][t], semi[b]).wait()
            pltpu.make_async_copy(whbm[t].at[pl.ds(r0, RCH * 16)], wb[b][t], semi[b]).wait()

    def fire_gathers(b):
        for t in range(4):
            pltpu.make_async_copy(table.at[ib[b][t]], gb[b][t], semg[b]).start()

    def wait_gathers(b):
        for t in range(4):
            pltpu.make_async_copy(table.at[ib[b][t]], gb[b][t], semg[b]).wait()

    def compute(it, b):
        r0 = base + it * RCH

        @pl.loop(0, RCH)
        def _(r):
            a0 = jnp.zeros((16,), jnp.float32)
            a1 = jnp.zeros((16,), jnp.float32)
            for j in range(16):
                k = r * 16 + j
                for t in range(4):
                    wv = plsc.load_gather(wb[b][t], [jnp.full((16,), k, jnp.int32)])
                    a0 = a0 + wv * gb[b][t][k, 0:16]
                    a1 = a1 + wv * gb[b][t][k, 16:32]
            ob[r, 0:16] = a0
            ob[r, 16:32] = a1

        pltpu.sync_copy(ob, out.at[pl.ds(r0, RCH)])

    fire_loads(0, 0)
    wait_loads(0, 0)
    fire_gathers(0)
    fire_loads(1, 1)

    @pl.loop(0, NIT // 2)
    def _(ii):
        for b in (0, 1):
            it = 2 * ii + b
            wait_gathers(b)

            @pl.when(it + 1 < NIT)
            def _():
                wait_loads(it + 1, 1 - b)
                fire_gathers(1 - b)

            compute(it, b)

            @pl.when(it + 2 < NIT)
            def _():
                fire_loads(it + 2, b)


@functools.lru_cache(maxsize=1)
def _get_sc_kernel():
    mesh = plsc.VectorSubcoreMesh(core_axis_name="c", subcore_axis_name="s")
    cp = pltpu.CompilerParams()
    fields = pltpu.CompilerParams.__dataclass_fields__
    if "needs_layout_passes" in fields:
        cp = dataclasses.replace(cp, needs_layout_passes=False)
    if "use_tc_tiling_on_sc" in fields:
        cp = dataclasses.replace(cp, use_tc_tiling_on_sc=False)
    return pl.kernel(
        _sc_body,
        mesh=mesh,
        compiler_params=cp,
        out_type=jax.ShapeDtypeStruct((NR, DH), jnp.float32),
        scratch_types=(
            [pltpu.VMEM((RCH * 16,), jnp.int32) for _ in range(8)]
            + [pltpu.VMEM((RCH * 16,), jnp.float32) for _ in range(8)]
            + [pltpu.VMEM((RCH * 16, DH), jnp.float32) for _ in range(8)]
            + [pltpu.VMEM((RCH, DH), jnp.float32)]
            + [pltpu.SemaphoreType.DMA] * 4
        ),
    )


def _sc_combine(value, idxs, wts):
    table = value.reshape(NR, DH)
    args = [table]
    for t in range(4):
        args.append(idxs[t].reshape(NR * 16))
    for t in range(4):
        args.append(wts[t].reshape(NR * 16))
    o = _get_sc_kernel()(*args)
    return o.reshape(BB, LQ, DM)


def _ln(x, g, b):
    mu = jnp.mean(x, axis=-1, keepdims=True)
    var = jnp.mean((x - mu) ** 2, axis=-1, keepdims=True)
    return (x - mu) * lax.rsqrt(var + 1e-5) * g + b


def _post_body(out_ref, o_ref, Wout_ref, bout_ref, g1_ref, be1_ref, W1_ref,
               b1_ref, W2_ref, b2_ref, g2_ref, be2_ref, new_ref):
    attn = jnp.dot(o_ref[0], Wout_ref[...], preferred_element_type=jnp.float32) + bout_ref[...]
    h1 = _ln(out_ref[0] + attn, g1_ref[...], be1_ref[...])
    a1 = jnp.maximum(jnp.dot(h1, W1_ref[...], preferred_element_type=jnp.float32) + b1_ref[...], 0.0)
    ffn = jnp.dot(a1, W2_ref[...], preferred_element_type=jnp.float32) + b2_ref[...]
    new_ref[0] = _ln(h1 + ffn, g2_ref[...], be2_ref[...])


def _post_layer(out, o, Wout, bout, g1, be1, W1, b1, W2, b2, g2, be2):
    return pl.pallas_call(
        _post_body,
        grid=(BB, NBLK),
        in_specs=[
            pl.BlockSpec((1, BLK, DM), lambda b, j: (b, j, 0)),
            pl.BlockSpec((1, BLK, DM), lambda b, j: (b, j, 0)),
            pl.BlockSpec((DM, DM), lambda b, j: (0, 0)),
            pl.BlockSpec((1, DM), lambda b, j: (0, 0)),
            pl.BlockSpec((1, DM), lambda b, j: (0, 0)),
            pl.BlockSpec((1, DM), lambda b, j: (0, 0)),
            pl.BlockSpec((DM, DF), lambda b, j: (0, 0)),
            pl.BlockSpec((1, DF), lambda b, j: (0, 0)),
            pl.BlockSpec((DF, DM), lambda b, j: (0, 0)),
            pl.BlockSpec((1, DM), lambda b, j: (0, 0)),
            pl.BlockSpec((1, DM), lambda b, j: (0, 0)),
            pl.BlockSpec((1, DM), lambda b, j: (0, 0)),
        ],
        out_specs=pl.BlockSpec((1, BLK, DM), lambda b, j: (b, j, 0)),
        out_shape=jax.ShapeDtypeStruct((BB, LQ, DM), jnp.float32),
    )(out, o, Wout, bout[None, :], g1[None, :], be1[None, :], W1, b1[None, :],
      W2, b2[None, :], g2[None, :], be2[None, :])


def kernel(src0, src1, src2, src3, pos0, pos1, pos2, pos3, level_embed, Wv, bv,
           Woff, boff, Wattn, battn, Wout, bout, g1, be1, W1, b1, W2, b2, g2, be2):
    srcs = [src0, src1, src2, src3]
    poss = [pos0, pos1, pos2, pos3]
    src_f = []
    pos_f = []
    for lvl in range(NL):
        b, c, h, w = srcs[lvl].shape
        src_f.append(srcs[lvl].reshape(b, c, h * w).transpose(0, 2, 1))
        pos_f.append(poss[lvl].reshape(b, c, h * w).transpose(0, 2, 1) + level_embed[lvl][None, None, :])
    src = jnp.concatenate(src_f, 1)
    pos = jnp.concatenate(pos_f, 1)

    Woffp = jnp.moveaxis(Woff.reshape(NLAY, DM, NH, NL, NP, 2), -1, 2).reshape(NLAY, DM, 2 * 128)
    boffp = jnp.moveaxis(boff.reshape(NLAY, NH, NL, NP, 2), -1, 1).reshape(NLAY, 2 * 128)

    out = src
    for i in range(NLAY):
        val, idxs, wts = _pre_layer(out, pos, Wv[i], bv[i], Woffp[i], boffp[i], Wattn[i], battn[i])
        o = _sc_combine(val, idxs, wts)
        out = _post_layer(out, o, Wout[i], bout[i], g1[i], be1[i], W1[i], b1[i], W2[i], b2[i], g2[i], be2[i])
    return out
